# Initial kernel scaffold; baseline (speedup 1.0000x reference)
import jax
import jax.numpy as jnp
from jax.experimental import pallas as pl


def kernel(target, source, feat):
    raise NotImplementedError("write your pallas kernel here")


# speedup vs baseline: 1.0513x; 1.0513x over previous
import functools

import jax
import jax.numpy as jnp
from jax import lax
from jax.experimental import pallas as pl
from jax.experimental.pallas import tpu as pltpu
from jax.experimental.pallas import tpu_sc as plsc

Q = 2048
K = 100000
QB = 512
SB = 1024
KPAD = 8
SPAD = 100352
NSB = SPAD // SB
NQB = Q // QB
TILES_PER_BLOCK = SB // 8
INT_MAX = 2**31 - 1
PAD_COORD = 1.0e15


def _argmin_body(tgt2t_ref, src_ref, w_ref, runval_ref, runtile_ref, t2_ref):
    qi = pl.program_id(0)
    si = pl.program_id(1)

    @pl.when(si == 0)
    def _init():
        tx = tgt2t_ref[0:1, :]
        ty = tgt2t_ref[1:2, :]
        tz = tgt2t_ref[2:3, :]
        t2_ref[...] = ((tx * tx + ty * ty) + tz * tz) * 0.25
        runval_ref[...] = jnp.full((8, QB), jnp.inf, dtype=jnp.float32)
        runtile_ref[...] = jnp.zeros((8, QB), dtype=jnp.int32)

    s = src_ref[...]
    sx = s[:, 0:1]
    sy = s[:, 1:2]
    sz = s[:, 2:3]
    s2 = (sx * sx + sy * sy) + sz * sz

    dot2 = lax.dot_general(
        s,
        tgt2t_ref[...],
        (((1,), (0,)), ((), ())),
        preferred_element_type=jnp.float32,
    )

    tps = t2_ref[...] + s2
    d2 = jnp.maximum(tps - dot2, 0.0)

    rv = runval_ref[...]
    rt = runtile_ref[...]
    base_tile = si * TILES_PER_BLOCK
    for st in range(TILES_PER_BLOCK):
        tile = d2[st * 8:(st + 1) * 8, :]
        m = tile < rv
        rv = jnp.where(m, tile, rv)
        rt = jnp.where(m, base_tile + st, rt)
    runval_ref[...] = rv
    runtile_ref[...] = rt

    @pl.when(si == NSB - 1)
    def _finish():
        rv_f = runval_ref[...]
        rt_f = runtile_ref[...]
        sub = lax.broadcasted_iota(jnp.int32, (8, QB), 0)
        sidx = rt_f * 8 + sub
        vmin = jnp.min(rv_f, axis=0, keepdims=True)
        cand = jnp.where(rv_f == vmin, sidx, INT_MAX)
        w_ref[...] = jnp.min(cand, axis=0, keepdims=True).reshape(1, 1, QB)


def _argmin_call(tgt2t, srcp):
    return pl.pallas_call(
        _argmin_body,
        grid=(NQB, NSB),
        in_specs=[
            pl.BlockSpec((KPAD, QB), lambda qi, si: (0, qi)),
            pl.BlockSpec((SB, KPAD), lambda qi, si: (si, 0)),
        ],
        out_specs=pl.BlockSpec((1, 1, QB), lambda qi, si: (qi, 0, 0)),
        out_shape=jax.ShapeDtypeStruct((NQB, 1, QB), jnp.int32),
        scratch_shapes=[
            pltpu.VMEM((8, QB), jnp.float32),
            pltpu.VMEM((8, QB), jnp.int32),
            pltpu.VMEM((1, QB), jnp.float32),
        ],
        compiler_params=pltpu.CompilerParams(
            dimension_semantics=("arbitrary", "arbitrary"),
        ),
    )(tgt2t, srcp)


IB = 256
NIB = Q // IB


def _unique_body(wcol_ref, wrow_ref, uniq_ref, uniq3_ref, first_ref, rank_ref):
    wcol = wcol_ref[...]
    wrow = wrow_ref[...]
    icol = lax.broadcasted_iota(jnp.int32, (Q, 1), 0)
    irow = lax.broadcasted_iota(jnp.int32, (1, Q), 1)

    for jb in range(NIB):
        wj = wrow[:, jb * IB:(jb + 1) * IB]
        jj = irow[:, jb * IB:(jb + 1) * IB]
        dup = jnp.logical_and(wcol == wj, icol < jj)
        cnt = jnp.sum(dup.astype(jnp.int32), axis=0, keepdims=True)
        first_ref[:, jb * IB:(jb + 1) * IB] = jnp.where(cnt == 0, 1, 0)

    first = first_ref[...]
    for ib in range(NIB):
        wi = wcol[ib * IB:(ib + 1) * IB, :]
        less = jnp.logical_and(first == 1, wrow < wi)
        rank_ref[ib * IB:(ib + 1) * IB, :] = jnp.sum(
            less.astype(jnp.int32), axis=1, keepdims=True
        )

    rank = rank_ref[...]
    for kb in range(NIB):
        krow = irow[:, kb * IB:(kb + 1) * IB]
        hit = rank == krow
        val = jnp.where(hit, wcol, -1)
        best = jnp.max(val, axis=0, keepdims=True)
        u = jnp.maximum(best, 0)
        uniq_ref[:, kb * IB:(kb + 1) * IB] = u
        ccol = lax.broadcasted_iota(jnp.int32, (3, 1), 0)
        uniq3_ref[:, kb * IB:(kb + 1) * IB] = u * 3 + ccol


def _unique_call(wcol, wrow):
    return pl.pallas_call(
        _unique_body,
        out_shape=[
            jax.ShapeDtypeStruct((1, Q), jnp.int32),
            jax.ShapeDtypeStruct((3, Q), jnp.int32),
        ],
        scratch_shapes=[
            pltpu.VMEM((1, Q), jnp.int32),
            pltpu.VMEM((Q, 1), jnp.int32),
        ],
    )(wcol, wrow)


NW = 32
ROWS_PER_W = Q // NW


def _sc_gather_body(feat_ref, srcflat_ref, uniq_ref, uniq3_ref,
                    outf_ref, outc_ref,
                    idxv, rows_f, idxc, vals_c, sem1, sem2):
    cid = lax.axis_index("c")
    sid = lax.axis_index("s")
    wid = sid * 2 + cid
    base = wid * ROWS_PER_W
    pltpu.sync_copy(uniq_ref.at[pl.ds(base, ROWS_PER_W)], idxv)
    cp1 = pltpu.async_copy(feat_ref.at[idxv], rows_f, sem1)
    for c in range(3):
        pltpu.sync_copy(uniq3_ref.at[c, pl.ds(base, ROWS_PER_W)], idxc)
        cp2 = pltpu.async_copy(srcflat_ref.at[idxc], vals_c, sem2)
        cp2.wait()
        pltpu.sync_copy(vals_c, outc_ref.at[c, pl.ds(base, ROWS_PER_W)])
    cp1.wait()
    pltpu.sync_copy(rows_f, outf_ref.at[pl.ds(base, ROWS_PER_W)])


def _sc_gather(feat, srcflat, uniq, uniq3):
    mesh = plsc.VectorSubcoreMesh(core_axis_name="c", subcore_axis_name="s")
    fn = functools.partial(
        pl.kernel,
        out_type=[
            jax.ShapeDtypeStruct((Q, 128), jnp.float32),
            jax.ShapeDtypeStruct((3, Q), jnp.float32),
        ],
        mesh=mesh,
        scratch_types=[
            pltpu.VMEM((ROWS_PER_W,), jnp.int32),
            pltpu.VMEM((ROWS_PER_W, 128), jnp.float32),
            pltpu.VMEM((ROWS_PER_W,), jnp.int32),
            pltpu.VMEM((ROWS_PER_W,), jnp.float32),
            pltpu.SemaphoreType.DMA,
            pltpu.SemaphoreType.DMA,
        ],
    )(_sc_gather_body)
    return fn(feat, srcflat, uniq, uniq3)


def kernel(target, source, feat):
    tgt2t = jnp.pad((2.0 * target).T, ((0, KPAD - 3), (0, 0)))
    srcp = jnp.pad(
        source, ((0, SPAD - K), (0, KPAD - 3)), constant_values=PAD_COORD
    )

    w4 = _argmin_call(tgt2t, srcp)
    wcol = w4.reshape(Q, 1)
    wrow = w4.reshape(1, Q)
    uniq_row, uniq3 = _unique_call(wcol, wrow)
    uniq = uniq_row.reshape(Q)

    srcflat = source.reshape(3 * K)
    outf, outc3 = _sc_gather(feat, srcflat, uniq, uniq3)
    return (outc3.T, outf)


# speedup vs baseline: 1.2086x; 1.1496x over previous
import functools

import jax
import jax.numpy as jnp
from jax import lax
from jax.experimental import pallas as pl
from jax.experimental.pallas import tpu as pltpu
from jax.experimental.pallas import tpu_sc as plsc

Q = 2048
K = 100000
QB = 512
SB = 1024
KPAD = 8
SPAD = 100352
NSB = SPAD // SB
NQB = Q // QB
TILES_PER_BLOCK = SB // 8
INT_MAX = 2**31 - 1
PAD_COORD = 1.0e15


NT = SB // 8


def _argmin_body(tgt2t_ref, src_ref, w_ref, runval_ref, runtile_ref,
                 t2c_ref, s2_ref):
    si = pl.program_id(0)
    qi = pl.program_id(1)

    @pl.when(qi == 0)
    def _per_source_block():
        s = src_ref[...]
        sx = s[:, 0:1]
        sy = s[:, 1:2]
        sz = s[:, 2:3]
        s2_ref[...] = (sx * sx + sy * sy) + sz * sz

    @pl.when(si == 0)
    def _per_query_block():
        t = tgt2t_ref[...]
        tx = t[0:1, :]
        ty = t[1:2, :]
        tz = t[2:3, :]
        t2c_ref[pl.ds(qi, 1), :] = ((tx * tx + ty * ty) + tz * tz) * 0.25
        runval_ref[pl.ds(qi * 8, 8), :] = jnp.full(
            (8, QB), jnp.inf, dtype=jnp.float32
        )
        runtile_ref[pl.ds(qi * 8, 8), :] = jnp.zeros((8, QB), dtype=jnp.int32)

    dot2 = lax.dot_general(
        src_ref[...],
        tgt2t_ref[...],
        (((1,), (0,)), ((), ())),
        preferred_element_type=jnp.float32,
    )

    tps = t2c_ref[pl.ds(qi, 1), :] + s2_ref[...]
    d2 = jnp.maximum(tps - dot2, 0.0)

    pairs = [(d2[t * 8:(t + 1) * 8, :], t) for t in range(NT)]
    while len(pairs) > 1:
        nxt = []
        for a in range(0, len(pairs), 2):
            (av, ai), (bv, bi) = pairs[a], pairs[a + 1]
            m = jnp.minimum(av, bv)
            if isinstance(ai, int):
                tid = jnp.where(bv < av, jnp.int32(bi), jnp.int32(ai))
            else:
                tid = jnp.where(bv < av, bi, ai)
            nxt.append((m, tid))
        pairs = nxt
    bestv, bestt = pairs[0]

    rv = runval_ref[pl.ds(qi * 8, 8), :]
    mask = bestv < rv
    runval_ref[pl.ds(qi * 8, 8), :] = jnp.where(mask, bestv, rv)
    runtile_ref[pl.ds(qi * 8, 8), :] = jnp.where(
        mask, si * NT + bestt, runtile_ref[pl.ds(qi * 8, 8), :]
    )

    @pl.when(si == NSB - 1)
    def _finish():
        rv_f = runval_ref[pl.ds(qi * 8, 8), :]
        rt_f = runtile_ref[pl.ds(qi * 8, 8), :]
        sub = lax.broadcasted_iota(jnp.int32, (8, QB), 0)
        sidx = rt_f * 8 + sub
        vmin = jnp.min(rv_f, axis=0, keepdims=True)
        cand = jnp.where(rv_f == vmin, sidx, INT_MAX)
        w_ref[...] = jnp.min(cand, axis=0, keepdims=True).reshape(1, 1, QB)


def _argmin_call(tgt2t, srcp):
    return pl.pallas_call(
        _argmin_body,
        grid=(NSB, NQB),
        in_specs=[
            pl.BlockSpec((KPAD, QB), lambda si, qi: (0, qi)),
            pl.BlockSpec((SB, KPAD), lambda si, qi: (si, 0)),
        ],
        out_specs=pl.BlockSpec((1, 1, QB), lambda si, qi: (qi, 0, 0)),
        out_shape=jax.ShapeDtypeStruct((NQB, 1, QB), jnp.int32),
        scratch_shapes=[
            pltpu.VMEM((NQB * 8, QB), jnp.float32),
            pltpu.VMEM((NQB * 8, QB), jnp.int32),
            pltpu.VMEM((NQB, QB), jnp.float32),
            pltpu.VMEM((SB, 1), jnp.float32),
        ],
        compiler_params=pltpu.CompilerParams(
            dimension_semantics=("arbitrary", "arbitrary"),
        ),
    )(tgt2t, srcp)


IB = 256
NIB = Q // IB


def _unique_body(wcol_ref, wrow_ref, uniq_ref, uniq3_ref, first_ref, rank_ref):
    wcol = wcol_ref[...]
    wrow = wrow_ref[...]
    icol = lax.broadcasted_iota(jnp.int32, (Q, 1), 0)
    irow = lax.broadcasted_iota(jnp.int32, (1, Q), 1)

    for jb in range(NIB):
        wj = wrow[:, jb * IB:(jb + 1) * IB]
        jj = irow[:, jb * IB:(jb + 1) * IB]
        dup = jnp.logical_and(wcol == wj, icol < jj)
        cnt = jnp.sum(dup.astype(jnp.int32), axis=0, keepdims=True)
        first_ref[:, jb * IB:(jb + 1) * IB] = jnp.where(cnt == 0, 1, 0)

    first = first_ref[...]
    for ib in range(NIB):
        wi = wcol[ib * IB:(ib + 1) * IB, :]
        less = jnp.logical_and(first == 1, wrow < wi)
        rank_ref[ib * IB:(ib + 1) * IB, :] = jnp.sum(
            less.astype(jnp.int32), axis=1, keepdims=True
        )

    rank = rank_ref[...]
    for kb in range(NIB):
        krow = irow[:, kb * IB:(kb + 1) * IB]
        hit = rank == krow
        val = jnp.where(hit, wcol, -1)
        best = jnp.max(val, axis=0, keepdims=True)
        u = jnp.maximum(best, 0)
        uniq_ref[:, kb * IB:(kb + 1) * IB] = u
        ccol = lax.broadcasted_iota(jnp.int32, (3, 1), 0)
        uniq3_ref[:, kb * IB:(kb + 1) * IB] = u * 3 + ccol


def _unique_call(wcol, wrow):
    return pl.pallas_call(
        _unique_body,
        out_shape=[
            jax.ShapeDtypeStruct((1, Q), jnp.int32),
            jax.ShapeDtypeStruct((3, Q), jnp.int32),
        ],
        scratch_shapes=[
            pltpu.VMEM((1, Q), jnp.int32),
            pltpu.VMEM((Q, 1), jnp.int32),
        ],
    )(wcol, wrow)


NW = 32
ROWS_PER_W = Q // NW


def _sc_gather_body(feat_ref, srcflat_ref, uniq_ref, uniq3_ref,
                    outf_ref, outc_ref,
                    idxv, rows_f, idxc, vals_c, sem1, sem2):
    cid = lax.axis_index("c")
    sid = lax.axis_index("s")
    wid = sid * 2 + cid
    base = wid * ROWS_PER_W
    pltpu.sync_copy(uniq_ref.at[pl.ds(base, ROWS_PER_W)], idxv)
    cp1 = pltpu.async_copy(feat_ref.at[idxv], rows_f, sem1)
    for c in range(3):
        pltpu.sync_copy(uniq3_ref.at[c, pl.ds(base, ROWS_PER_W)], idxc)
        cp2 = pltpu.async_copy(srcflat_ref.at[idxc], vals_c, sem2)
        cp2.wait()
        pltpu.sync_copy(vals_c, outc_ref.at[c, pl.ds(base, ROWS_PER_W)])
    cp1.wait()
    pltpu.sync_copy(rows_f, outf_ref.at[pl.ds(base, ROWS_PER_W)])


def _sc_gather(feat, srcflat, uniq, uniq3):
    mesh = plsc.VectorSubcoreMesh(core_axis_name="c", subcore_axis_name="s")
    fn = functools.partial(
        pl.kernel,
        out_type=[
            jax.ShapeDtypeStruct((Q, 128), jnp.float32),
            jax.ShapeDtypeStruct((3, Q), jnp.float32),
        ],
        mesh=mesh,
        scratch_types=[
            pltpu.VMEM((ROWS_PER_W,), jnp.int32),
            pltpu.VMEM((ROWS_PER_W, 128), jnp.float32),
            pltpu.VMEM((ROWS_PER_W,), jnp.int32),
            pltpu.VMEM((ROWS_PER_W,), jnp.float32),
            pltpu.SemaphoreType.DMA,
            pltpu.SemaphoreType.DMA,
        ],
    )(_sc_gather_body)
    return fn(feat, srcflat, uniq, uniq3)


def kernel(target, source, feat):
    tgt2t = jnp.pad((2.0 * target).T, ((0, KPAD - 3), (0, 0)))
    srcp = jnp.pad(
        source, ((0, SPAD - K), (0, KPAD - 3)), constant_values=PAD_COORD
    )

    w4 = _argmin_call(tgt2t, srcp)
    wcol = w4.reshape(Q, 1)
    wrow = w4.reshape(1, Q)
    uniq_row, uniq3 = _unique_call(wcol, wrow)
    uniq = uniq_row.reshape(Q)

    srcflat = source.reshape(3 * K)
    outf, outc3 = _sc_gather(feat, srcflat, uniq, uniq3)
    return (outc3.T, outf)


# speedup vs baseline: 1.6499x; 1.3652x over previous
import functools

import jax
import jax.numpy as jnp
from jax import lax
from jax.experimental import pallas as pl
from jax.experimental.pallas import tpu as pltpu
from jax.experimental.pallas import tpu_sc as plsc

Q = 2048
K = 100000
QB = 512
SB = 2048
KPAD = 8
SPAD = 100352
NSB = SPAD // SB
NQB = Q // QB
TILES_PER_BLOCK = SB // 8
INT_MAX = 2**31 - 1
PAD_COORD = 1.0e15


NT = SB // 8


def _argmin_body(tgt2t_ref, src_ref, w_ref, runval_ref, runtile_ref,
                 t2c_ref, s2_ref):
    si = pl.program_id(0)
    qi = pl.program_id(1)

    @pl.when(qi == 0)
    def _per_source_block():
        s = src_ref[...]
        sx = s[:, 0:1]
        sy = s[:, 1:2]
        sz = s[:, 2:3]
        s2_ref[...] = (sx * sx + sy * sy) + sz * sz

    @pl.when(si == 0)
    def _per_query_block():
        t = tgt2t_ref[...]
        tx = t[0:1, :]
        ty = t[1:2, :]
        tz = t[2:3, :]
        t2c_ref[pl.ds(qi, 1), :] = ((tx * tx + ty * ty) + tz * tz) * 0.25
        runval_ref[pl.ds(qi * 8, 8), :] = jnp.full(
            (8, QB), jnp.inf, dtype=jnp.float32
        )
        runtile_ref[pl.ds(qi * 8, 8), :] = jnp.zeros((8, QB), dtype=jnp.int32)

    dot2 = lax.dot_general(
        src_ref[...],
        tgt2t_ref[...],
        (((1,), (0,)), ((), ())),
        preferred_element_type=jnp.float32,
    )

    tps = t2c_ref[pl.ds(qi, 1), :] + s2_ref[...]
    d2 = jnp.maximum(tps - dot2, 0.0)

    pairs = [(d2[t * 8:(t + 1) * 8, :], t) for t in range(NT)]
    while len(pairs) > 1:
        nxt = []
        for a in range(0, len(pairs), 2):
            (av, ai), (bv, bi) = pairs[a], pairs[a + 1]
            m = jnp.minimum(av, bv)
            if isinstance(ai, int):
                tid = jnp.where(bv < av, jnp.int32(bi), jnp.int32(ai))
            else:
                tid = jnp.where(bv < av, bi, ai)
            nxt.append((m, tid))
        pairs = nxt
    bestv, bestt = pairs[0]

    rv = runval_ref[pl.ds(qi * 8, 8), :]
    mask = bestv < rv
    runval_ref[pl.ds(qi * 8, 8), :] = jnp.where(mask, bestv, rv)
    runtile_ref[pl.ds(qi * 8, 8), :] = jnp.where(
        mask, si * NT + bestt, runtile_ref[pl.ds(qi * 8, 8), :]
    )

    @pl.when(si == NSB - 1)
    def _finish():
        rv_f = runval_ref[pl.ds(qi * 8, 8), :]
        rt_f = runtile_ref[pl.ds(qi * 8, 8), :]
        sub = lax.broadcasted_iota(jnp.int32, (8, QB), 0)
        sidx = rt_f * 8 + sub
        vmin = jnp.min(rv_f, axis=0, keepdims=True)
        cand = jnp.where(rv_f == vmin, sidx, INT_MAX)
        w_ref[...] = jnp.min(cand, axis=0, keepdims=True).reshape(1, 1, QB)


def _argmin_call(tgt2t, srcp):
    return pl.pallas_call(
        _argmin_body,
        grid=(NSB, NQB),
        in_specs=[
            pl.BlockSpec((KPAD, QB), lambda si, qi: (0, qi)),
            pl.BlockSpec((SB, KPAD), lambda si, qi: (si, 0)),
        ],
        out_specs=pl.BlockSpec((1, 1, QB), lambda si, qi: (qi, 0, 0)),
        out_shape=jax.ShapeDtypeStruct((NQB, 1, QB), jnp.int32),
        scratch_shapes=[
            pltpu.VMEM((NQB * 8, QB), jnp.float32),
            pltpu.VMEM((NQB * 8, QB), jnp.int32),
            pltpu.VMEM((NQB, QB), jnp.float32),
            pltpu.VMEM((SB, 1), jnp.float32),
        ],
        compiler_params=pltpu.CompilerParams(
            dimension_semantics=("arbitrary", "arbitrary"),
        ),
    )(tgt2t, srcp)


IB = 256
NIB = Q // IB


def _unique_body(wcol_ref, wrow_ref, uniq_ref, uniq3_ref, first_ref, rank_ref):
    wcol = wcol_ref[...]
    wrow = wrow_ref[...]
    icol = lax.broadcasted_iota(jnp.int32, (Q, 1), 0)
    irow = lax.broadcasted_iota(jnp.int32, (1, Q), 1)

    for jb in range(NIB):
        wj = wrow[:, jb * IB:(jb + 1) * IB]
        jj = irow[:, jb * IB:(jb + 1) * IB]
        dup = jnp.logical_and(wcol == wj, icol < jj)
        cnt = jnp.sum(dup.astype(jnp.int32), axis=0, keepdims=True)
        first_ref[:, jb * IB:(jb + 1) * IB] = jnp.where(cnt == 0, 1, 0)

    first = first_ref[...]
    for ib in range(NIB):
        wi = wcol[ib * IB:(ib + 1) * IB, :]
        less = jnp.logical_and(first == 1, wrow < wi)
        rank_ref[ib * IB:(ib + 1) * IB, :] = jnp.sum(
            less.astype(jnp.int32), axis=1, keepdims=True
        )

    rank = rank_ref[...]
    for kb in range(NIB):
        krow = irow[:, kb * IB:(kb + 1) * IB]
        hit = rank == krow
        val = jnp.where(hit, wcol, -1)
        best = jnp.max(val, axis=0, keepdims=True)
        u = jnp.maximum(best, 0)
        uniq_ref[:, kb * IB:(kb + 1) * IB] = u
        ccol = lax.broadcasted_iota(jnp.int32, (3, 1), 0)
        uniq3_ref[:, kb * IB:(kb + 1) * IB] = u + ccol * K


def _unique_call(wcol, wrow):
    return pl.pallas_call(
        _unique_body,
        out_shape=[
            jax.ShapeDtypeStruct((1, Q), jnp.int32),
            jax.ShapeDtypeStruct((3, Q), jnp.int32),
        ],
        scratch_shapes=[
            pltpu.VMEM((1, Q), jnp.int32),
            pltpu.VMEM((Q, 1), jnp.int32),
        ],
    )(wcol, wrow)


NW = 32
ROWS_PER_W = Q // NW


def _sc_gather_body(feat_ref, srcflat_ref, uniq_ref, uniq3_ref,
                    outf_ref, outc_ref,
                    idxv, rows_f, idxc, vals_c, sem1, sem2):
    cid = lax.axis_index("c")
    sid = lax.axis_index("s")
    wid = sid * 2 + cid
    base = wid * ROWS_PER_W
    pltpu.sync_copy(uniq_ref.at[pl.ds(base, ROWS_PER_W)], idxv)
    cp1 = pltpu.async_copy(feat_ref.at[idxv], rows_f, sem1)
    for c in range(3):
        pltpu.sync_copy(uniq3_ref.at[c, pl.ds(base, ROWS_PER_W)], idxc)
        cp2 = pltpu.async_copy(srcflat_ref.at[idxc], vals_c, sem2)
        cp2.wait()
        pltpu.sync_copy(vals_c, outc_ref.at[c, pl.ds(base, ROWS_PER_W)])
    cp1.wait()
    pltpu.sync_copy(rows_f, outf_ref.at[pl.ds(base, ROWS_PER_W)])


def _sc_gather(feat, srcflat, uniq, uniq3):
    mesh = plsc.VectorSubcoreMesh(core_axis_name="c", subcore_axis_name="s")
    fn = functools.partial(
        pl.kernel,
        out_type=[
            jax.ShapeDtypeStruct((Q, 128), jnp.float32),
            jax.ShapeDtypeStruct((3, Q), jnp.float32),
        ],
        mesh=mesh,
        scratch_types=[
            pltpu.VMEM((ROWS_PER_W,), jnp.int32),
            pltpu.VMEM((ROWS_PER_W, 128), jnp.float32),
            pltpu.VMEM((ROWS_PER_W,), jnp.int32),
            pltpu.VMEM((ROWS_PER_W,), jnp.float32),
            pltpu.SemaphoreType.DMA,
            pltpu.SemaphoreType.DMA,
        ],
    )(_sc_gather_body)
    return fn(feat, srcflat, uniq, uniq3)


def kernel(target, source, feat):
    tgt2t = jnp.pad((2.0 * target).T, ((0, KPAD - 3), (0, 0)))
    srcp = jnp.pad(
        source, ((0, SPAD - K), (0, KPAD - 3)), constant_values=PAD_COORD
    )

    w4 = _argmin_call(tgt2t, srcp)
    wcol = w4.reshape(Q, 1)
    wrow = w4.reshape(1, Q)
    uniq_row, uniq3 = _unique_call(wcol, wrow)
    uniq = uniq_row.reshape(Q)

    srcflat = source.T.reshape(3 * K)
    outf, outc3 = _sc_gather(feat, srcflat, uniq, uniq3)
    return (outc3.T, outf)


# speedup vs baseline: 1.7959x; 1.0885x over previous
import functools

import jax
import jax.numpy as jnp
from jax import lax
from jax.experimental import pallas as pl
from jax.experimental.pallas import tpu as pltpu
from jax.experimental.pallas import tpu_sc as plsc

Q = 2048
K = 100000
QB = 512
SB = 3584
KPAD = 8
SPAD = 100352
NSB = SPAD // SB
NQB = Q // QB
TILES_PER_BLOCK = SB // 8
INT_MAX = 2**31 - 1
PAD_COORD = 1.0e15


NT = SB // 8


def _argmin_body(tgt2t_ref, src_ref, w_ref, runval_ref, runtile_ref,
                 t2c_ref, s2_ref):
    si = pl.program_id(0)
    qi = pl.program_id(1)

    @pl.when(qi == 0)
    def _per_source_block():
        s = src_ref[...]
        sx = s[:, 0:1]
        sy = s[:, 1:2]
        sz = s[:, 2:3]
        s2_ref[...] = (sx * sx + sy * sy) + sz * sz

    @pl.when(si == 0)
    def _per_query_block():
        t = tgt2t_ref[...]
        tx = t[0:1, :]
        ty = t[1:2, :]
        tz = t[2:3, :]
        t2c_ref[pl.ds(qi, 1), :] = ((tx * tx + ty * ty) + tz * tz) * 0.25
        runval_ref[pl.ds(qi * 8, 8), :] = jnp.full(
            (8, QB), jnp.inf, dtype=jnp.float32
        )
        runtile_ref[pl.ds(qi * 8, 8), :] = jnp.zeros((8, QB), dtype=jnp.int32)

    dot2 = lax.dot_general(
        src_ref[...],
        tgt2t_ref[...],
        (((1,), (0,)), ((), ())),
        preferred_element_type=jnp.float32,
    )

    tps = t2c_ref[pl.ds(qi, 1), :] + s2_ref[...]
    d2 = jnp.maximum(tps - dot2, 0.0)

    pairs = [(d2[t * 8:(t + 1) * 8, :], t) for t in range(NT)]
    while len(pairs) > 1:
        nxt = []
        if len(pairs) % 2:
            carry = [pairs[-1]]
        else:
            carry = []
        for a in range(0, len(pairs) - 1, 2):
            (av, ai), (bv, bi) = pairs[a], pairs[a + 1]
            m = jnp.minimum(av, bv)
            if isinstance(ai, int):
                tid = jnp.where(bv < av, jnp.int32(bi), jnp.int32(ai))
            else:
                tid = jnp.where(bv < av, bi, ai)
            nxt.append((m, tid))
        pairs = nxt + carry
    bestv, bestt = pairs[0]

    rv = runval_ref[pl.ds(qi * 8, 8), :]
    mask = bestv < rv
    runval_ref[pl.ds(qi * 8, 8), :] = jnp.where(mask, bestv, rv)
    runtile_ref[pl.ds(qi * 8, 8), :] = jnp.where(
        mask, si * NT + bestt, runtile_ref[pl.ds(qi * 8, 8), :]
    )

    @pl.when(si == NSB - 1)
    def _finish():
        rv_f = runval_ref[pl.ds(qi * 8, 8), :]
        rt_f = runtile_ref[pl.ds(qi * 8, 8), :]
        sub = lax.broadcasted_iota(jnp.int32, (8, QB), 0)
        sidx = rt_f * 8 + sub
        vmin = jnp.min(rv_f, axis=0, keepdims=True)
        cand = jnp.where(rv_f == vmin, sidx, INT_MAX)
        w_ref[...] = jnp.min(cand, axis=0, keepdims=True).reshape(1, 1, QB)


def _argmin_call(tgt2t, srcp):
    return pl.pallas_call(
        _argmin_body,
        grid=(NSB, NQB),
        in_specs=[
            pl.BlockSpec((KPAD, QB), lambda si, qi: (0, qi)),
            pl.BlockSpec((SB, KPAD), lambda si, qi: (si, 0)),
        ],
        out_specs=pl.BlockSpec((1, 1, QB), lambda si, qi: (qi, 0, 0)),
        out_shape=jax.ShapeDtypeStruct((NQB, 1, QB), jnp.int32),
        scratch_shapes=[
            pltpu.VMEM((NQB * 8, QB), jnp.float32),
            pltpu.VMEM((NQB * 8, QB), jnp.int32),
            pltpu.VMEM((NQB, QB), jnp.float32),
            pltpu.VMEM((SB, 1), jnp.float32),
        ],
        compiler_params=pltpu.CompilerParams(
            dimension_semantics=("arbitrary", "arbitrary"),
        ),
    )(tgt2t, srcp)


IB = 256
NIB = Q // IB


def _unique_body(wcol_ref, wrow_ref, uniq_ref, uniq3_ref, first_ref, rank_ref):
    wcol = wcol_ref[...]
    wrow = wrow_ref[...]
    icol = lax.broadcasted_iota(jnp.int32, (Q, 1), 0)
    irow = lax.broadcasted_iota(jnp.int32, (1, Q), 1)

    for jb in range(NIB):
        wj = wrow[:, jb * IB:(jb + 1) * IB]
        jj = irow[:, jb * IB:(jb + 1) * IB]
        dup = jnp.logical_and(wcol == wj, icol < jj)
        cnt = jnp.sum(dup.astype(jnp.int32), axis=0, keepdims=True)
        first_ref[:, jb * IB:(jb + 1) * IB] = jnp.where(cnt == 0, 1, 0)

    first = first_ref[...]
    for ib in range(NIB):
        wi = wcol[ib * IB:(ib + 1) * IB, :]
        less = jnp.logical_and(first == 1, wrow < wi)
        rank_ref[ib * IB:(ib + 1) * IB, :] = jnp.sum(
            less.astype(jnp.int32), axis=1, keepdims=True
        )

    rank = rank_ref[...]
    for kb in range(NIB):
        krow = irow[:, kb * IB:(kb + 1) * IB]
        hit = rank == krow
        val = jnp.where(hit, wcol, -1)
        best = jnp.max(val, axis=0, keepdims=True)
        u = jnp.maximum(best, 0)
        uniq_ref[:, kb * IB:(kb + 1) * IB] = u
        ccol = lax.broadcasted_iota(jnp.int32, (3, 1), 0)
        uniq3_ref[:, kb * IB:(kb + 1) * IB] = u + ccol * K


def _unique_call(wcol, wrow):
    return pl.pallas_call(
        _unique_body,
        out_shape=[
            jax.ShapeDtypeStruct((1, Q), jnp.int32),
            jax.ShapeDtypeStruct((3, Q), jnp.int32),
        ],
        scratch_shapes=[
            pltpu.VMEM((1, Q), jnp.int32),
            pltpu.VMEM((Q, 1), jnp.int32),
        ],
    )(wcol, wrow)


NW = 32
ROWS_PER_W = Q // NW


def _sc_gather_body(feat_ref, srcflat_ref, uniq_ref, uniq3_ref,
                    outf_ref, outc_ref,
                    idxv, rows_f, idxc, vals_c, sem1, sem2):
    cid = lax.axis_index("c")
    sid = lax.axis_index("s")
    wid = sid * 2 + cid
    base = wid * ROWS_PER_W
    pltpu.sync_copy(uniq_ref.at[pl.ds(base, ROWS_PER_W)], idxv)
    cp1 = pltpu.async_copy(feat_ref.at[idxv], rows_f, sem1)
    for c in range(3):
        pltpu.sync_copy(uniq3_ref.at[c, pl.ds(base, ROWS_PER_W)], idxc)
        cp2 = pltpu.async_copy(srcflat_ref.at[idxc], vals_c, sem2)
        cp2.wait()
        pltpu.sync_copy(vals_c, outc_ref.at[c, pl.ds(base, ROWS_PER_W)])
    cp1.wait()
    pltpu.sync_copy(rows_f, outf_ref.at[pl.ds(base, ROWS_PER_W)])


def _sc_gather(feat, srcflat, uniq, uniq3):
    mesh = plsc.VectorSubcoreMesh(core_axis_name="c", subcore_axis_name="s")
    fn = functools.partial(
        pl.kernel,
        out_type=[
            jax.ShapeDtypeStruct((Q, 128), jnp.float32),
            jax.ShapeDtypeStruct((3, Q), jnp.float32),
        ],
        mesh=mesh,
        scratch_types=[
            pltpu.VMEM((ROWS_PER_W,), jnp.int32),
            pltpu.VMEM((ROWS_PER_W, 128), jnp.float32),
            pltpu.VMEM((ROWS_PER_W,), jnp.int32),
            pltpu.VMEM((ROWS_PER_W,), jnp.float32),
            pltpu.SemaphoreType.DMA,
            pltpu.SemaphoreType.DMA,
        ],
    )(_sc_gather_body)
    return fn(feat, srcflat, uniq, uniq3)


def kernel(target, source, feat):
    tgt2t = jnp.pad((2.0 * target).T, ((0, KPAD - 3), (0, 0)))
    srcp = jnp.pad(
        source, ((0, SPAD - K), (0, KPAD - 3)), constant_values=PAD_COORD
    )

    w4 = _argmin_call(tgt2t, srcp)
    wcol = w4.reshape(Q, 1)
    wrow = w4.reshape(1, Q)
    uniq_row, uniq3 = _unique_call(wcol, wrow)
    uniq = uniq_row.reshape(Q)

    srcflat = source.T.reshape(3 * K)
    outf, outc3 = _sc_gather(feat, srcflat, uniq, uniq3)
    return (outc3.T, outf)


# speedup vs baseline: 1.8568x; 1.0339x over previous
import functools

import jax
import jax.numpy as jnp
from jax import lax
from jax.experimental import pallas as pl
from jax.experimental.pallas import tpu as pltpu
from jax.experimental.pallas import tpu_sc as plsc

Q = 2048
K = 100000
QB = 512
SB = 7168
KPAD = 8
SPAD = 100352
NSB = SPAD // SB
NQB = Q // QB
TILES_PER_BLOCK = SB // 8
INT_MAX = 2**31 - 1
PAD_COORD = 1.0e15


NT = SB // 8


def _argmin_body(tgt2t_ref, src_ref, w_ref, runval_ref, runtile_ref,
                 t2c_ref, s2_ref):
    si = pl.program_id(0)
    qi = pl.program_id(1)

    @pl.when(qi == 0)
    def _per_source_block():
        s = src_ref[...]
        sx = s[:, 0:1]
        sy = s[:, 1:2]
        sz = s[:, 2:3]
        s2_ref[...] = (sx * sx + sy * sy) + sz * sz

    @pl.when(si == 0)
    def _per_query_block():
        t = tgt2t_ref[...]
        tx = t[0:1, :]
        ty = t[1:2, :]
        tz = t[2:3, :]
        t2c_ref[pl.ds(qi, 1), :] = ((tx * tx + ty * ty) + tz * tz) * 0.25
        runval_ref[pl.ds(qi * 8, 8), :] = jnp.full(
            (8, QB), jnp.inf, dtype=jnp.float32
        )
        runtile_ref[pl.ds(qi * 8, 8), :] = jnp.zeros((8, QB), dtype=jnp.int32)

    dot2 = lax.dot_general(
        src_ref[...],
        tgt2t_ref[...],
        (((1,), (0,)), ((), ())),
        preferred_element_type=jnp.float32,
    )

    tps = t2c_ref[pl.ds(qi, 1), :] + s2_ref[...]
    d2 = jnp.maximum(tps - dot2, 0.0)

    pairs = [(d2[t * 8:(t + 1) * 8, :], t) for t in range(NT)]
    while len(pairs) > 1:
        nxt = []
        if len(pairs) % 2:
            carry = [pairs[-1]]
        else:
            carry = []
        for a in range(0, len(pairs) - 1, 2):
            (av, ai), (bv, bi) = pairs[a], pairs[a + 1]
            m = jnp.minimum(av, bv)
            if isinstance(ai, int):
                tid = jnp.where(bv < av, jnp.int32(bi), jnp.int32(ai))
            else:
                tid = jnp.where(bv < av, bi, ai)
            nxt.append((m, tid))
        pairs = nxt + carry
    bestv, bestt = pairs[0]

    rv = runval_ref[pl.ds(qi * 8, 8), :]
    mask = bestv < rv
    runval_ref[pl.ds(qi * 8, 8), :] = jnp.where(mask, bestv, rv)
    runtile_ref[pl.ds(qi * 8, 8), :] = jnp.where(
        mask, si * NT + bestt, runtile_ref[pl.ds(qi * 8, 8), :]
    )

    @pl.when(si == NSB - 1)
    def _finish():
        rv_f = runval_ref[pl.ds(qi * 8, 8), :]
        rt_f = runtile_ref[pl.ds(qi * 8, 8), :]
        sub = lax.broadcasted_iota(jnp.int32, (8, QB), 0)
        sidx = rt_f * 8 + sub
        vmin = jnp.min(rv_f, axis=0, keepdims=True)
        cand = jnp.where(rv_f == vmin, sidx, INT_MAX)
        w_ref[...] = jnp.min(cand, axis=0, keepdims=True).reshape(1, 1, QB)


def _argmin_call(tgt2t, srcp):
    return pl.pallas_call(
        _argmin_body,
        grid=(NSB, NQB),
        in_specs=[
            pl.BlockSpec((KPAD, QB), lambda si, qi: (0, qi)),
            pl.BlockSpec((SB, KPAD), lambda si, qi: (si, 0)),
        ],
        out_specs=pl.BlockSpec((1, 1, QB), lambda si, qi: (qi, 0, 0)),
        out_shape=jax.ShapeDtypeStruct((NQB, 1, QB), jnp.int32),
        scratch_shapes=[
            pltpu.VMEM((NQB * 8, QB), jnp.float32),
            pltpu.VMEM((NQB * 8, QB), jnp.int32),
            pltpu.VMEM((NQB, QB), jnp.float32),
            pltpu.VMEM((SB, 1), jnp.float32),
        ],
        compiler_params=pltpu.CompilerParams(
            dimension_semantics=("arbitrary", "arbitrary"),
        ),
    )(tgt2t, srcp)


IB = 256
NIB = Q // IB


def _unique_body(wcol_ref, wrow_ref, uniq_ref, uniq3_ref, first_ref, rank_ref):
    wcol = wcol_ref[...]
    wrow = wrow_ref[...]
    icol = lax.broadcasted_iota(jnp.int32, (Q, 1), 0)
    irow = lax.broadcasted_iota(jnp.int32, (1, Q), 1)

    for jb in range(NIB):
        wj = wrow[:, jb * IB:(jb + 1) * IB]
        jj = irow[:, jb * IB:(jb + 1) * IB]
        dup = jnp.logical_and(wcol == wj, icol < jj)
        cnt = jnp.sum(dup.astype(jnp.int32), axis=0, keepdims=True)
        first_ref[:, jb * IB:(jb + 1) * IB] = jnp.where(cnt == 0, 1, 0)

    first = first_ref[...]
    for ib in range(NIB):
        wi = wcol[ib * IB:(ib + 1) * IB, :]
        less = jnp.logical_and(first == 1, wrow < wi)
        rank_ref[ib * IB:(ib + 1) * IB, :] = jnp.sum(
            less.astype(jnp.int32), axis=1, keepdims=True
        )

    rank = rank_ref[...]
    for kb in range(NIB):
        krow = irow[:, kb * IB:(kb + 1) * IB]
        hit = rank == krow
        val = jnp.where(hit, wcol, -1)
        best = jnp.max(val, axis=0, keepdims=True)
        u = jnp.maximum(best, 0)
        uniq_ref[:, kb * IB:(kb + 1) * IB] = u
        ccol = lax.broadcasted_iota(jnp.int32, (3, 1), 0)
        uniq3_ref[:, kb * IB:(kb + 1) * IB] = u + ccol * K


def _unique_call(wcol, wrow):
    return pl.pallas_call(
        _unique_body,
        out_shape=[
            jax.ShapeDtypeStruct((1, Q), jnp.int32),
            jax.ShapeDtypeStruct((3, Q), jnp.int32),
        ],
        scratch_shapes=[
            pltpu.VMEM((1, Q), jnp.int32),
            pltpu.VMEM((Q, 1), jnp.int32),
        ],
    )(wcol, wrow)


NW = 32
ROWS_PER_W = Q // NW


def _sc_gather_body(feat_ref, srcflat_ref, uniq_ref, uniq3_ref,
                    outf_ref, outc_ref,
                    idxv, rows_f, idxc, vals_c, sem1, sem2):
    cid = lax.axis_index("c")
    sid = lax.axis_index("s")
    wid = sid * 2 + cid
    base = wid * ROWS_PER_W
    pltpu.sync_copy(uniq_ref.at[pl.ds(base, ROWS_PER_W)], idxv)
    cp1 = pltpu.async_copy(feat_ref.at[idxv], rows_f, sem1)
    for c in range(3):
        pltpu.sync_copy(uniq3_ref.at[c, pl.ds(base, ROWS_PER_W)], idxc)
        cp2 = pltpu.async_copy(srcflat_ref.at[idxc], vals_c, sem2)
        cp2.wait()
        pltpu.sync_copy(vals_c, outc_ref.at[c, pl.ds(base, ROWS_PER_W)])
    cp1.wait()
    pltpu.sync_copy(rows_f, outf_ref.at[pl.ds(base, ROWS_PER_W)])


def _sc_gather(feat, srcflat, uniq, uniq3):
    mesh = plsc.VectorSubcoreMesh(core_axis_name="c", subcore_axis_name="s")
    fn = functools.partial(
        pl.kernel,
        out_type=[
            jax.ShapeDtypeStruct((Q, 128), jnp.float32),
            jax.ShapeDtypeStruct((3, Q), jnp.float32),
        ],
        mesh=mesh,
        scratch_types=[
            pltpu.VMEM((ROWS_PER_W,), jnp.int32),
            pltpu.VMEM((ROWS_PER_W, 128), jnp.float32),
            pltpu.VMEM((ROWS_PER_W,), jnp.int32),
            pltpu.VMEM((ROWS_PER_W,), jnp.float32),
            pltpu.SemaphoreType.DMA,
            pltpu.SemaphoreType.DMA,
        ],
    )(_sc_gather_body)
    return fn(feat, srcflat, uniq, uniq3)


def kernel(target, source, feat):
    tgt2t = jnp.pad((2.0 * target).T, ((0, KPAD - 3), (0, 0)))
    srcp = jnp.pad(
        source, ((0, SPAD - K), (0, KPAD - 3)), constant_values=PAD_COORD
    )

    w4 = _argmin_call(tgt2t, srcp)
    wcol = w4.reshape(Q, 1)
    wrow = w4.reshape(1, Q)
    uniq_row, uniq3 = _unique_call(wcol, wrow)
    uniq = uniq_row.reshape(Q)

    srcflat = source.T.reshape(3 * K)
    outf, outc3 = _sc_gather(feat, srcflat, uniq, uniq3)
    return (outc3.T, outf)


# speedup vs baseline: 1.9014x; 1.0240x over previous
import functools

import jax
import jax.numpy as jnp
from jax import lax
from jax.experimental import pallas as pl
from jax.experimental.pallas import tpu as pltpu
from jax.experimental.pallas import tpu_sc as plsc

Q = 2048
K = 100000
QB = 1024
SB = 7168
KPAD = 8
SPAD = 100352
NSB = SPAD // SB
NQB = Q // QB
TILES_PER_BLOCK = SB // 8
INT_MAX = 2**31 - 1
PAD_COORD = 1.0e15


NT = SB // 8


def _argmin_body(tgt2t_ref, src_ref, w_ref, runval_ref, runtile_ref,
                 t2c_ref, s2_ref):
    si = pl.program_id(0)
    qi = pl.program_id(1)

    @pl.when(qi == 0)
    def _per_source_block():
        s = src_ref[...]
        sx = s[:, 0:1]
        sy = s[:, 1:2]
        sz = s[:, 2:3]
        s2_ref[...] = (sx * sx + sy * sy) + sz * sz

    @pl.when(si == 0)
    def _per_query_block():
        t = tgt2t_ref[...]
        tx = t[0:1, :]
        ty = t[1:2, :]
        tz = t[2:3, :]
        t2c_ref[pl.ds(qi, 1), :] = ((tx * tx + ty * ty) + tz * tz) * 0.25
        runval_ref[pl.ds(qi * 8, 8), :] = jnp.full(
            (8, QB), jnp.inf, dtype=jnp.float32
        )
        runtile_ref[pl.ds(qi * 8, 8), :] = jnp.zeros((8, QB), dtype=jnp.int32)

    dot2 = lax.dot_general(
        src_ref[...],
        tgt2t_ref[...],
        (((1,), (0,)), ((), ())),
        preferred_element_type=jnp.float32,
    )

    tps = t2c_ref[pl.ds(qi, 1), :] + s2_ref[...]
    d2 = jnp.maximum(tps - dot2, 0.0)

    pairs = [(d2[t * 8:(t + 1) * 8, :], t) for t in range(NT)]
    while len(pairs) > 1:
        nxt = []
        if len(pairs) % 2:
            carry = [pairs[-1]]
        else:
            carry = []
        for a in range(0, len(pairs) - 1, 2):
            (av, ai), (bv, bi) = pairs[a], pairs[a + 1]
            m = jnp.minimum(av, bv)
            if isinstance(ai, int):
                tid = jnp.where(bv < av, jnp.int32(bi), jnp.int32(ai))
            else:
                tid = jnp.where(bv < av, bi, ai)
            nxt.append((m, tid))
        pairs = nxt + carry
    bestv, bestt = pairs[0]

    rv = runval_ref[pl.ds(qi * 8, 8), :]
    mask = bestv < rv
    runval_ref[pl.ds(qi * 8, 8), :] = jnp.where(mask, bestv, rv)
    runtile_ref[pl.ds(qi * 8, 8), :] = jnp.where(
        mask, si * NT + bestt, runtile_ref[pl.ds(qi * 8, 8), :]
    )

    @pl.when(si == NSB - 1)
    def _finish():
        rv_f = runval_ref[pl.ds(qi * 8, 8), :]
        rt_f = runtile_ref[pl.ds(qi * 8, 8), :]
        sub = lax.broadcasted_iota(jnp.int32, (8, QB), 0)
        sidx = rt_f * 8 + sub
        vmin = jnp.min(rv_f, axis=0, keepdims=True)
        cand = jnp.where(rv_f == vmin, sidx, INT_MAX)
        w_ref[...] = jnp.min(cand, axis=0, keepdims=True).reshape(1, 1, QB)


def _argmin_call(tgt2t, srcp):
    return pl.pallas_call(
        _argmin_body,
        grid=(NSB, NQB),
        in_specs=[
            pl.BlockSpec((KPAD, QB), lambda si, qi: (0, qi)),
            pl.BlockSpec((SB, KPAD), lambda si, qi: (si, 0)),
        ],
        out_specs=pl.BlockSpec((1, 1, QB), lambda si, qi: (qi, 0, 0)),
        out_shape=jax.ShapeDtypeStruct((NQB, 1, QB), jnp.int32),
        scratch_shapes=[
            pltpu.VMEM((NQB * 8, QB), jnp.float32),
            pltpu.VMEM((NQB * 8, QB), jnp.int32),
            pltpu.VMEM((NQB, QB), jnp.float32),
            pltpu.VMEM((SB, 1), jnp.float32),
        ],
        compiler_params=pltpu.CompilerParams(
            dimension_semantics=("arbitrary", "arbitrary"),
        ),
    )(tgt2t, srcp)


IB = 256
NIB = Q // IB


def _unique_body(wcol_ref, wrow_ref, uniq_ref, uniq3_ref, first_ref, rank_ref):
    wcol = wcol_ref[...]
    wrow = wrow_ref[...]
    icol = lax.broadcasted_iota(jnp.int32, (Q, 1), 0)
    irow = lax.broadcasted_iota(jnp.int32, (1, Q), 1)

    for jb in range(NIB):
        wj = wrow[:, jb * IB:(jb + 1) * IB]
        jj = irow[:, jb * IB:(jb + 1) * IB]
        dup = jnp.logical_and(wcol == wj, icol < jj)
        cnt = jnp.sum(dup.astype(jnp.int32), axis=0, keepdims=True)
        first_ref[:, jb * IB:(jb + 1) * IB] = jnp.where(cnt == 0, 1, 0)

    first = first_ref[...]
    for ib in range(NIB):
        wi = wcol[ib * IB:(ib + 1) * IB, :]
        less = jnp.logical_and(first == 1, wrow < wi)
        rank_ref[ib * IB:(ib + 1) * IB, :] = jnp.sum(
            less.astype(jnp.int32), axis=1, keepdims=True
        )

    rank = rank_ref[...]
    for kb in range(NIB):
        krow = irow[:, kb * IB:(kb + 1) * IB]
        hit = rank == krow
        val = jnp.where(hit, wcol, -1)
        best = jnp.max(val, axis=0, keepdims=True)
        u = jnp.maximum(best, 0)
        uniq_ref[:, kb * IB:(kb + 1) * IB] = u
        ccol = lax.broadcasted_iota(jnp.int32, (3, 1), 0)
        uniq3_ref[:, kb * IB:(kb + 1) * IB] = u + ccol * K


def _unique_call(wcol, wrow):
    return pl.pallas_call(
        _unique_body,
        out_shape=[
            jax.ShapeDtypeStruct((1, Q), jnp.int32),
            jax.ShapeDtypeStruct((3, Q), jnp.int32),
        ],
        scratch_shapes=[
            pltpu.VMEM((1, Q), jnp.int32),
            pltpu.VMEM((Q, 1), jnp.int32),
        ],
    )(wcol, wrow)


NW = 32
ROWS_PER_W = Q // NW


def _sc_gather_body(feat_ref, srcflat_ref, uniq_ref, uniq3_ref,
                    outf_ref, outc_ref,
                    idxv, rows_f, idxc, vals_c, sem1, sem2):
    cid = lax.axis_index("c")
    sid = lax.axis_index("s")
    wid = sid * 2 + cid
    base = wid * ROWS_PER_W
    pltpu.sync_copy(uniq_ref.at[pl.ds(base, ROWS_PER_W)], idxv)
    cp1 = pltpu.async_copy(feat_ref.at[idxv], rows_f, sem1)
    for c in range(3):
        pltpu.sync_copy(uniq3_ref.at[c, pl.ds(base, ROWS_PER_W)], idxc)
        cp2 = pltpu.async_copy(srcflat_ref.at[idxc], vals_c, sem2)
        cp2.wait()
        pltpu.sync_copy(vals_c, outc_ref.at[c, pl.ds(base, ROWS_PER_W)])
    cp1.wait()
    pltpu.sync_copy(rows_f, outf_ref.at[pl.ds(base, ROWS_PER_W)])


def _sc_gather(feat, srcflat, uniq, uniq3):
    mesh = plsc.VectorSubcoreMesh(core_axis_name="c", subcore_axis_name="s")
    fn = functools.partial(
        pl.kernel,
        out_type=[
            jax.ShapeDtypeStruct((Q, 128), jnp.float32),
            jax.ShapeDtypeStruct((3, Q), jnp.float32),
        ],
        mesh=mesh,
        scratch_types=[
            pltpu.VMEM((ROWS_PER_W,), jnp.int32),
            pltpu.VMEM((ROWS_PER_W, 128), jnp.float32),
            pltpu.VMEM((ROWS_PER_W,), jnp.int32),
            pltpu.VMEM((ROWS_PER_W,), jnp.float32),
            pltpu.SemaphoreType.DMA,
            pltpu.SemaphoreType.DMA,
        ],
    )(_sc_gather_body)
    return fn(feat, srcflat, uniq, uniq3)


def kernel(target, source, feat):
    tgt2t = jnp.pad((2.0 * target).T, ((0, KPAD - 3), (0, 0)))
    srcp = jnp.pad(
        source, ((0, SPAD - K), (0, KPAD - 3)), constant_values=PAD_COORD
    )

    w4 = _argmin_call(tgt2t, srcp)
    wcol = w4.reshape(Q, 1)
    wrow = w4.reshape(1, Q)
    uniq_row, uniq3 = _unique_call(wcol, wrow)
    uniq = uniq_row.reshape(Q)

    srcflat = source.T.reshape(3 * K)
    outf, outc3 = _sc_gather(feat, srcflat, uniq, uniq3)
    return (outc3.T, outf)


# speedup vs baseline: 1.9027x; 1.0007x over previous
import functools

import jax
import jax.numpy as jnp
from jax import lax
from jax.experimental import pallas as pl
from jax.experimental.pallas import tpu as pltpu
from jax.experimental.pallas import tpu_sc as plsc

Q = 2048
K = 100000
QB = 1024
SB = 7168
KPAD = 8
SPAD = 100352
NSB = SPAD // SB
NQB = Q // QB
INT_MAX = 2**31 - 1
PAD_COORD = 1.0e15


NT = SB // 8


def _argmin_body(tgt2t_ref, src_ref, w_ref, runval_ref, runtile_ref,
                 t2c_ref, s2_ref):
    si = pl.program_id(0)
    qi = pl.program_id(1)

    @pl.when(qi == 0)
    def _per_source_block():
        s = src_ref[...]
        sx = s[:, 0:1]
        sy = s[:, 1:2]
        sz = s[:, 2:3]
        s2_ref[...] = (sx * sx + sy * sy) + sz * sz

    @pl.when(si == 0)
    def _per_query_block():
        t = tgt2t_ref[...]
        tx = t[0:1, :]
        ty = t[1:2, :]
        tz = t[2:3, :]
        t2c_ref[pl.ds(qi, 1), :] = ((tx * tx + ty * ty) + tz * tz) * 0.25
        runval_ref[pl.ds(qi * 8, 8), :] = jnp.full(
            (8, QB), jnp.inf, dtype=jnp.float32
        )
        runtile_ref[pl.ds(qi * 8, 8), :] = jnp.zeros((8, QB), dtype=jnp.int32)

    dot2 = lax.dot_general(
        src_ref[...],
        tgt2t_ref[...],
        (((1,), (0,)), ((), ())),
        preferred_element_type=jnp.float32,
    )

    tps = t2c_ref[pl.ds(qi, 1), :] + s2_ref[...]
    d2 = jnp.maximum(tps - dot2, 0.0)

    pairs = [(d2[t * 8:(t + 1) * 8, :], t) for t in range(NT)]
    while len(pairs) > 1:
        nxt = []
        if len(pairs) % 2:
            carry = [pairs[-1]]
        else:
            carry = []
        for a in range(0, len(pairs) - 1, 2):
            (av, ai), (bv, bi) = pairs[a], pairs[a + 1]
            m = jnp.minimum(av, bv)
            if isinstance(ai, int):
                tid = jnp.where(bv < av, jnp.int32(bi), jnp.int32(ai))
            else:
                tid = jnp.where(bv < av, bi, ai)
            nxt.append((m, tid))
        pairs = nxt + carry
    bestv, bestt = pairs[0]

    rv = runval_ref[pl.ds(qi * 8, 8), :]
    mask = bestv < rv
    runval_ref[pl.ds(qi * 8, 8), :] = jnp.where(mask, bestv, rv)
    runtile_ref[pl.ds(qi * 8, 8), :] = jnp.where(
        mask, si * NT + bestt, runtile_ref[pl.ds(qi * 8, 8), :]
    )

    @pl.when(si == NSB - 1)
    def _finish():
        rv_f = runval_ref[pl.ds(qi * 8, 8), :]
        rt_f = runtile_ref[pl.ds(qi * 8, 8), :]
        sub = lax.broadcasted_iota(jnp.int32, (8, QB), 0)
        sidx = rt_f * 8 + sub
        vmin = jnp.min(rv_f, axis=0, keepdims=True)
        cand = jnp.where(rv_f == vmin, sidx, INT_MAX)
        w_ref[...] = jnp.min(cand, axis=0, keepdims=True).reshape(1, 1, QB)


def _argmin_call(tgt2t, srcp):
    return pl.pallas_call(
        _argmin_body,
        grid=(NSB, NQB),
        in_specs=[
            pl.BlockSpec((KPAD, QB), lambda si, qi: (0, qi)),
            pl.BlockSpec((SB, KPAD), lambda si, qi: (si, 0)),
        ],
        out_specs=pl.BlockSpec((1, 1, QB), lambda si, qi: (qi, 0, 0)),
        out_shape=jax.ShapeDtypeStruct((NQB, 1, QB), jnp.int32),
        scratch_shapes=[
            pltpu.VMEM((NQB * 8, QB), jnp.float32),
            pltpu.VMEM((NQB * 8, QB), jnp.int32),
            pltpu.VMEM((NQB, QB), jnp.float32),
            pltpu.VMEM((SB, 1), jnp.float32),
        ],
        compiler_params=pltpu.CompilerParams(
            dimension_semantics=("arbitrary", "arbitrary"),
        ),
    )(tgt2t, srcp)


IB = 256
NIB = Q // IB


def _unique_body(wcol_ref, wrow_ref, uniq_ref, uniq3_ref, first_ref, rank_ref):
    wcol = wcol_ref[...]
    wrow = wrow_ref[...]
    icol = lax.broadcasted_iota(jnp.int32, (Q, 1), 0)
    irow = lax.broadcasted_iota(jnp.int32, (1, Q), 1)

    for jb in range(NIB):
        wj = wrow[:, jb * IB:(jb + 1) * IB]
        jj = irow[:, jb * IB:(jb + 1) * IB]
        dup = jnp.logical_and(wcol == wj, icol < jj)
        cnt = jnp.sum(dup.astype(jnp.int32), axis=0, keepdims=True)
        first_ref[:, jb * IB:(jb + 1) * IB] = jnp.where(cnt == 0, 1, 0)

    first = first_ref[...]
    for ib in range(NIB):
        wi = wcol[ib * IB:(ib + 1) * IB, :]
        less = jnp.logical_and(first == 1, wrow < wi)
        rank_ref[ib * IB:(ib + 1) * IB, :] = jnp.sum(
            less.astype(jnp.int32), axis=1, keepdims=True
        )

    rank = rank_ref[...]
    for kb in range(NIB):
        krow = irow[:, kb * IB:(kb + 1) * IB]
        hit = rank == krow
        val = jnp.where(hit, wcol, -1)
        best = jnp.max(val, axis=0, keepdims=True)
        u = jnp.maximum(best, 0)
        uniq_ref[:, kb * IB:(kb + 1) * IB] = u
        ccol = lax.broadcasted_iota(jnp.int32, (3, 1), 0)
        uniq3_ref[:, kb * IB:(kb + 1) * IB] = u + ccol * K


def _unique_call(wcol, wrow):
    return pl.pallas_call(
        _unique_body,
        out_shape=[
            jax.ShapeDtypeStruct((1, Q), jnp.int32),
            jax.ShapeDtypeStruct((3, Q), jnp.int32),
        ],
        scratch_shapes=[
            pltpu.VMEM((1, Q), jnp.int32),
            pltpu.VMEM((Q, 1), jnp.int32),
        ],
    )(wcol, wrow)


NW = 32
ROWS_PER_W = Q // NW


def _sc_gather_body(feat_ref, srcflat_ref, uniq_ref, uniq3_ref,
                    outf_ref, outc_ref,
                    idxv, rows_f, idxc, vals_c, sem1, sem2):
    cid = lax.axis_index("c")
    sid = lax.axis_index("s")
    wid = sid * 2 + cid
    base = wid * ROWS_PER_W
    pltpu.sync_copy(uniq_ref.at[pl.ds(base, ROWS_PER_W)], idxv)
    cp1 = pltpu.async_copy(feat_ref.at[idxv], rows_f, sem1)
    for c in range(3):
        pltpu.sync_copy(uniq3_ref.at[c, pl.ds(base, ROWS_PER_W)], idxc)
        cp2 = pltpu.async_copy(srcflat_ref.at[idxc], vals_c, sem2)
        cp2.wait()
        pltpu.sync_copy(vals_c, outc_ref.at[c, pl.ds(base, ROWS_PER_W)])
    cp1.wait()
    pltpu.sync_copy(rows_f, outf_ref.at[pl.ds(base, ROWS_PER_W)])


def _sc_gather(feat, srcflat, uniq, uniq3):
    mesh = plsc.VectorSubcoreMesh(core_axis_name="c", subcore_axis_name="s")
    fn = functools.partial(
        pl.kernel,
        out_type=[
            jax.ShapeDtypeStruct((Q, 128), jnp.float32),
            jax.ShapeDtypeStruct((3, Q), jnp.float32),
        ],
        mesh=mesh,
        scratch_types=[
            pltpu.VMEM((ROWS_PER_W,), jnp.int32),
            pltpu.VMEM((ROWS_PER_W, 128), jnp.float32),
            pltpu.VMEM((ROWS_PER_W,), jnp.int32),
            pltpu.VMEM((ROWS_PER_W,), jnp.float32),
            pltpu.SemaphoreType.DMA,
            pltpu.SemaphoreType.DMA,
        ],
    )(_sc_gather_body)
    return fn(feat, srcflat, uniq, uniq3)


def kernel(target, source, feat):
    tgt2t = jnp.pad((2.0 * target).T, ((0, KPAD - 3), (0, 0)))
    srcp = jnp.pad(
        source, ((0, SPAD - K), (0, KPAD - 3)), constant_values=PAD_COORD
    )

    w4 = _argmin_call(tgt2t, srcp)
    wcol = w4.reshape(Q, 1)
    wrow = w4.reshape(1, Q)
    uniq_row, uniq3 = _unique_call(wcol, wrow)
    uniq = uniq_row.reshape(Q)

    srcflat = source.T.reshape(3 * K)
    outf, outc3 = _sc_gather(feat, srcflat, uniq, uniq3)
    return (outc3.T, outf)
